# baseline (speedup 1.0000x reference)
import jax
import jax.numpy as jnp
import numpy as np
from jax.experimental import pallas as pl
from jax.experimental.pallas import tpu as pltpu

_S = 32


def _gibbs_body(z0n_ref, z0p_ref, u_ref, pip_ref, byp_ref, lpp_ref,
                e2_ref, em2_ref, btT_ref, b0_ref, yo_ref, out_ref, e_ref,
                cop_ref, fp_ref):
    S, CP, NB = e_ref.shape
    M = z0p_ref.shape[0]

    z0f = z0n_ref[...].reshape(S * NB, M)
    lin0 = jnp.dot(z0f, btT_ref[...], preferred_element_type=jnp.float32)
    lin0 = (lin0 + b0_ref[...]).reshape(S, NB, CP)
    e0 = jnp.exp(lin0)
    e_ref[...] = jnp.transpose(e0, (0, 2, 1))
    d0 = jnp.sum(e0, axis=2)
    ly0 = jnp.sum(lin0 * yo_ref[...][None], axis=2)
    t3acc0 = jnp.zeros((S, NB), jnp.float32)

    def step_body(t, carry):
        d, ly, t3acc = carry
        pij = pip_ref[pl.ds(t, 1), :]
        bjy = byp_ref[pl.ds(t, 1), :]
        e2r = e2_ref[pl.ds(t, 1), :]
        em2r = em2_ref[pl.ds(t, 1), :]
        e2c = jnp.broadcast_to(e2r.T, (CP, NB))
        em2c = jnp.broadcast_to(em2r.T, (CP, NB))

        for s in range(S):
            zrow = z0p_ref[t, pl.ds(s, 1), :]
            es = e_ref[s]
            gs = jnp.where(zrow > 0.0, em2c, e2c)
            cop_ref[pl.ds(s, 1), :] = jnp.sum(es * gs, axis=0, keepdims=True)

        zj = z0p_ref[t]
        u = u_ref[t]
        copp = cop_ref[...]
        zpos = zj > 0.0
        dpos = jnp.where(zpos, d, copp)
        dneg = jnp.where(zpos, copp, d)
        npos = jnp.exp(ly + (1.0 - zj) * bjy)
        nneg = jnp.exp(ly + (-1.0 - zj) * bjy)
        gpos = npos / dpos
        gneg = nneg / dneg
        num = gpos * pij
        p = num / (num + gneg * (1.0 - pij))
        znew = jnp.where(u < p, 1.0, -1.0)

        flip = znew != zj
        fp_ref[...] = znew * zj
        d_new = jnp.where(flip, copp, d)
        ly_new = ly + (znew - zj) * bjy
        t3acc = t3acc + (znew + 1.0) * 0.5 * lpp_ref[pl.ds(t, 1), :]

        for s in range(S):
            zrow = z0p_ref[t, pl.ds(s, 1), :]
            frow = fp_ref[pl.ds(s, 1), :]
            es = e_ref[s]
            gs = jnp.where(zrow > 0.0, em2c, e2c)
            e_ref[s] = es * jnp.where(frow < 0.0, gs, 1.0)
        return d_new, ly_new, t3acc

    d, ly, t3acc = jax.lax.fori_loop(0, M, step_body, (d0, ly0, t3acc0))

    t1 = jnp.sum(ly) * (1.0 / S)
    t2 = jnp.sum(jnp.log(d)) * (1.0 / S)
    t3 = jnp.sum(t3acc) * (1.0 / S)
    lp = lpp_ref[...]
    t4 = jnp.sum(jnp.maximum(lp, 0.0) + jnp.log1p(jnp.exp(-jnp.abs(lp))))
    out_ref[...] = jnp.full((1, 1, 128), t1 - t2 + t3 - t4, jnp.float32)


def kernel(x, y, alpha_0, alpha, beta_0, beta):
    S = _S
    N, _F = x.shape
    C, M = beta.shape
    CP = 128
    NB = 128
    nblk = N // NB

    linear_pi = alpha_0[None, :] + x @ alpha.T
    pi_pos = jax.nn.sigmoid(linear_pi)
    key = jax.random.key(42)
    k_init, k_gibbs = jax.random.split(key)
    z0 = 2.0 * jax.random.bernoulli(k_init, pi_pos, (S, N, M)).astype(x.dtype) - 1.0
    order = np.random.RandomState(0).permutation(M)
    keys = jax.random.split(k_gibbs, M)
    u = jax.vmap(lambda k: jax.random.uniform(k, (S, N), jnp.float32))(keys)

    z0p = jnp.transpose(z0, (2, 0, 1))[order]
    pip = pi_pos.T[order]
    byp = beta[y, :].T[order]
    lpp = linear_pi.T[order]
    btp = jnp.pad(beta.T[order], ((0, 0), (0, CP - C)))
    e2 = jnp.exp(2.0 * btp)
    em2 = jnp.exp(-2.0 * btp)
    btT = jnp.pad(beta.T, ((0, 0), (0, CP - C)))
    b0 = jnp.full((1, CP), -1e30, jnp.float32).at[0, :C].set(beta_0)
    yo = jax.nn.one_hot(y, CP, dtype=jnp.float32)

    devs = jax.devices()
    nd = 2 if len(devs) >= 2 else 1
    mesh = jax.sharding.Mesh(np.array(devs[:nd]), ("d",))
    P = jax.sharding.PartitionSpec

    def _run(z0_, z0p_, u_, pip_, byp_, lpp_, e2_, em2_, btT_, b0_, yo_):
        nb_loc = z0_.shape[1] // NB
        return pl.pallas_call(
            _gibbs_body,
            out_shape=jax.ShapeDtypeStruct((nb_loc, 1, 128), jnp.float32),
            grid=(nb_loc,),
            in_specs=[
                pl.BlockSpec((S, NB, M), lambda i: (0, i, 0)),
                pl.BlockSpec((M, S, NB), lambda i: (0, 0, i)),
                pl.BlockSpec((M, S, NB), lambda i: (0, 0, i)),
                pl.BlockSpec((M, NB), lambda i: (0, i)),
                pl.BlockSpec((M, NB), lambda i: (0, i)),
                pl.BlockSpec((M, NB), lambda i: (0, i)),
                pl.BlockSpec((M, CP), lambda i: (0, 0)),
                pl.BlockSpec((M, CP), lambda i: (0, 0)),
                pl.BlockSpec((M, CP), lambda i: (0, 0)),
                pl.BlockSpec((1, CP), lambda i: (0, 0)),
                pl.BlockSpec((NB, CP), lambda i: (i, 0)),
            ],
            out_specs=pl.BlockSpec((1, 1, 128), lambda i: (i, 0, 0)),
            scratch_shapes=[pltpu.VMEM((S, CP, NB), jnp.float32),
                            pltpu.VMEM((S, NB), jnp.float32),
                            pltpu.VMEM((S, NB), jnp.float32)],
            compiler_params=pltpu.CompilerParams(
                dimension_semantics=("parallel",),
                vmem_limit_bytes=48 * 1024 * 1024,
            ),
            name="lb_cnn_gibbs",
            interpret=False,
        )(z0_, z0p_, u_, pip_, byp_, lpp_, e2_, em2_, btT_, b0_, yo_)

    sharded = jax.shard_map(
        _run,
        mesh=mesh,
        in_specs=(
            P(None, "d", None),
            P(None, None, "d"),
            P(None, None, "d"),
            P(None, "d"),
            P(None, "d"),
            P(None, "d"),
            P(None, None),
            P(None, None),
            P(None, None),
            P(None, None),
            P("d", None),
        ),
        out_specs=P("d", None, None),
        check_vma=False,
    )
    out = sharded(z0, z0p, u, pip, byp, lpp, e2, em2, btT, b0, yo)
    return jnp.sum(out[:, 0, 0])


# speedup vs baseline: 1.0469x; 1.0469x over previous
import jax
import jax.numpy as jnp
import numpy as np
from jax.experimental import pallas as pl
from jax.experimental.pallas import tpu as pltpu

_S = 32


def _gibbs_body(z0n_ref, z0p_ref, u_ref, pip_ref, byp_ref, lpp_ref,
                e2_ref, em2_ref, btT_ref, b0_ref, yo_ref, out_ref, e_ref):
    S, CP, NB = e_ref.shape
    M = z0p_ref.shape[0]

    z0f = z0n_ref[...].reshape(S * NB, M)
    lin0 = jnp.dot(z0f, btT_ref[...], preferred_element_type=jnp.float32)
    lin0 = (lin0 + b0_ref[...]).reshape(S, NB, CP)
    e0 = jnp.exp(lin0)
    e_ref[...] = jnp.transpose(e0, (0, 2, 1))
    d0 = jnp.sum(e0, axis=2)
    ly0 = jnp.sum(lin0 * yo_ref[...][None], axis=2)
    t3acc0 = jnp.zeros((S, NB), jnp.float32)

    def step_body(t, carry):
        d, ly, t3acc = carry
        e = e_ref[...]
        zj = z0p_ref[t]
        u = u_ref[t]
        pij = pip_ref[pl.ds(t, 1), :]
        bjy = byp_ref[pl.ds(t, 1), :]
        e2r = e2_ref[pl.ds(t, 1), :]
        em2r = em2_ref[pl.ds(t, 1), :]
        e2c = jnp.broadcast_to(e2r.T, (CP, NB))
        em2c = jnp.broadcast_to(em2r.T, (CP, NB))

        zpos = zj > 0.0
        gsel = jnp.where(zj[:, None, :] > 0.0, em2c[None], e2c[None])
        copp = jnp.sum(e * gsel, axis=1)
        dpos = jnp.where(zpos, d, copp)
        dneg = jnp.where(zpos, copp, d)
        npos = jnp.exp(ly + (1.0 - zj) * bjy)
        nneg = jnp.exp(ly + (-1.0 - zj) * bjy)
        gpos = npos / dpos
        gneg = nneg / dneg
        num = gpos * pij
        p = num / (num + gneg * (1.0 - pij))
        znew = jnp.where(u < p, 1.0, -1.0)

        flip = znew != zj
        fprod = znew * zj
        e_ref[...] = e * jnp.where(fprod[:, None, :] < 0.0, gsel, 1.0)
        d_new = jnp.where(flip, copp, d)
        ly_new = ly + (znew - zj) * bjy
        t3acc = t3acc + (znew + 1.0) * 0.5 * lpp_ref[pl.ds(t, 1), :]
        return d_new, ly_new, t3acc

    def step8(i, carry):
        for k in range(8):
            carry = step_body(8 * i + k, carry)
        return carry

    d, ly, t3acc = jax.lax.fori_loop(0, M // 8, step8, (d0, ly0, t3acc0))

    t1 = jnp.sum(ly) * (1.0 / S)
    t2 = jnp.sum(jnp.log(d)) * (1.0 / S)
    t3 = jnp.sum(t3acc) * (1.0 / S)
    lp = lpp_ref[...]
    t4 = jnp.sum(jnp.maximum(lp, 0.0) + jnp.log1p(jnp.exp(-jnp.abs(lp))))
    out_ref[...] = jnp.full((1, 1, 128), t1 - t2 + t3 - t4, jnp.float32)


def kernel(x, y, alpha_0, alpha, beta_0, beta):
    S = _S
    N, _F = x.shape
    C, M = beta.shape
    CP = 128
    NB = 128
    nblk = N // NB

    linear_pi = alpha_0[None, :] + x @ alpha.T
    pi_pos = jax.nn.sigmoid(linear_pi)
    key = jax.random.key(42)
    k_init, k_gibbs = jax.random.split(key)
    z0 = 2.0 * jax.random.bernoulli(k_init, pi_pos, (S, N, M)).astype(x.dtype) - 1.0
    order = np.random.RandomState(0).permutation(M)
    keys = jax.random.split(k_gibbs, M)
    u = jax.vmap(lambda k: jax.random.uniform(k, (S, N), jnp.float32))(keys)

    z0p = jnp.transpose(z0, (2, 0, 1))[order]
    pip = pi_pos.T[order]
    byp = beta[y, :].T[order]
    lpp = linear_pi.T[order]
    btp = jnp.pad(beta.T[order], ((0, 0), (0, CP - C)))
    e2 = jnp.exp(2.0 * btp)
    em2 = jnp.exp(-2.0 * btp)
    btT = jnp.pad(beta.T, ((0, 0), (0, CP - C)))
    b0 = jnp.full((1, CP), -1e30, jnp.float32).at[0, :C].set(beta_0)
    yo = jax.nn.one_hot(y, CP, dtype=jnp.float32)

    devs = jax.devices()
    nd = 2 if len(devs) >= 2 else 1
    mesh = jax.sharding.Mesh(np.array(devs[:nd]), ("d",))
    P = jax.sharding.PartitionSpec

    def _run(z0_, z0p_, u_, pip_, byp_, lpp_, e2_, em2_, btT_, b0_, yo_):
        nb_loc = z0_.shape[1] // NB
        return pl.pallas_call(
            _gibbs_body,
            out_shape=jax.ShapeDtypeStruct((nb_loc, 1, 128), jnp.float32),
            grid=(nb_loc,),
            in_specs=[
                pl.BlockSpec((S, NB, M), lambda i: (0, i, 0)),
                pl.BlockSpec((M, S, NB), lambda i: (0, 0, i)),
                pl.BlockSpec((M, S, NB), lambda i: (0, 0, i)),
                pl.BlockSpec((M, NB), lambda i: (0, i)),
                pl.BlockSpec((M, NB), lambda i: (0, i)),
                pl.BlockSpec((M, NB), lambda i: (0, i)),
                pl.BlockSpec((M, CP), lambda i: (0, 0)),
                pl.BlockSpec((M, CP), lambda i: (0, 0)),
                pl.BlockSpec((M, CP), lambda i: (0, 0)),
                pl.BlockSpec((1, CP), lambda i: (0, 0)),
                pl.BlockSpec((NB, CP), lambda i: (i, 0)),
            ],
            out_specs=pl.BlockSpec((1, 1, 128), lambda i: (i, 0, 0)),
            scratch_shapes=[pltpu.VMEM((S, CP, NB), jnp.float32)],
            compiler_params=pltpu.CompilerParams(
                dimension_semantics=("parallel",),
                vmem_limit_bytes=48 * 1024 * 1024,
            ),
            name="lb_cnn_gibbs",
            interpret=False,
        )(z0_, z0p_, u_, pip_, byp_, lpp_, e2_, em2_, btT_, b0_, yo_)

    sharded = jax.shard_map(
        _run,
        mesh=mesh,
        in_specs=(
            P(None, "d", None),
            P(None, None, "d"),
            P(None, None, "d"),
            P(None, "d"),
            P(None, "d"),
            P(None, "d"),
            P(None, None),
            P(None, None),
            P(None, None),
            P(None, None),
            P("d", None),
        ),
        out_specs=P("d", None, None),
        check_vma=False,
    )
    out = sharded(z0, z0p, u, pip, byp, lpp, e2, em2, btT, b0, yo)
    return jnp.sum(out[:, 0, 0])


# speedup vs baseline: 1.0530x; 1.0059x over previous
import jax
import jax.numpy as jnp
import numpy as np
from jax.experimental import pallas as pl
from jax.experimental.pallas import tpu as pltpu

_S = 32


def _gibbs_body(z0n_ref, z0p_ref, u_ref, pip_ref, byp_ref, lpp_ref,
                e2_ref, em2_ref, btT_ref, b0_ref, yo_ref, out_ref, e_ref):
    S, CP, NB = e_ref.shape
    M = z0p_ref.shape[0]

    z0f = z0n_ref[...].reshape(S * NB, M)
    lin0 = jnp.dot(z0f, btT_ref[...], preferred_element_type=jnp.float32)
    lin0 = (lin0 + b0_ref[...]).reshape(S, NB, CP)
    e0 = jnp.exp(lin0)
    e_ref[...] = jnp.transpose(e0, (0, 2, 1))
    d0 = jnp.sum(e0, axis=2)
    ly0 = jnp.sum(lin0 * yo_ref[...][None], axis=2)
    t3acc0 = jnp.zeros((S, NB), jnp.float32)

    def step_body(t, carry):
        d, ly, t3acc = carry
        e = e_ref[...]
        zj = z0p_ref[t]
        u = u_ref[t]
        pij = pip_ref[pl.ds(t, 1), :]
        bjy = byp_ref[pl.ds(t, 1), :]
        e2r = e2_ref[pl.ds(t, 1), :]
        em2r = em2_ref[pl.ds(t, 1), :]
        e2c = jnp.broadcast_to(e2r.T, (CP, NB))
        em2c = jnp.broadcast_to(em2r.T, (CP, NB))

        zpos = zj > 0.0
        gsel = jnp.where(zj[:, None, :] > 0.0, em2c[None], e2c[None])
        copp = jnp.sum(e * gsel, axis=1)
        dpos = jnp.where(zpos, d, copp)
        dneg = jnp.where(zpos, copp, d)
        npos = jnp.exp(ly + (1.0 - zj) * bjy)
        nneg = jnp.exp(ly + (-1.0 - zj) * bjy)
        gpos = npos / dpos
        gneg = nneg / dneg
        num = gpos * pij
        p = num / (num + gneg * (1.0 - pij))
        znew = jnp.where(u < p, 1.0, -1.0)

        flip = znew != zj
        fprod = znew * zj
        e_ref[...] = e * jnp.where(fprod[:, None, :] < 0.0, gsel, 1.0)
        d_new = jnp.where(flip, copp, d)
        ly_new = ly + (znew - zj) * bjy
        t3acc = t3acc + (znew + 1.0) * 0.5 * lpp_ref[pl.ds(t, 1), :]
        return d_new, ly_new, t3acc

    def step4(i, carry):
        for k in range(4):
            carry = step_body(4 * i + k, carry)
        return carry

    d, ly, t3acc = jax.lax.fori_loop(0, M // 4, step4, (d0, ly0, t3acc0))

    t1 = jnp.sum(ly) * (1.0 / S)
    t2 = jnp.sum(jnp.log(d)) * (1.0 / S)
    t3 = jnp.sum(t3acc) * (1.0 / S)
    lp = lpp_ref[...]
    t4 = jnp.sum(jnp.maximum(lp, 0.0) + jnp.log1p(jnp.exp(-jnp.abs(lp))))
    out_ref[...] = jnp.full((1, 1, 128), t1 - t2 + t3 - t4, jnp.float32)


def kernel(x, y, alpha_0, alpha, beta_0, beta):
    S = _S
    N, _F = x.shape
    C, M = beta.shape
    CP = 128
    NB = 128
    nblk = N // NB

    linear_pi = alpha_0[None, :] + x @ alpha.T
    pi_pos = jax.nn.sigmoid(linear_pi)
    key = jax.random.key(42)
    k_init, k_gibbs = jax.random.split(key)
    z0 = 2.0 * jax.random.bernoulli(k_init, pi_pos, (S, N, M)).astype(x.dtype) - 1.0
    order = np.random.RandomState(0).permutation(M)
    keys = jax.random.split(k_gibbs, M)
    u = jax.vmap(lambda k: jax.random.uniform(k, (S, N), jnp.float32))(keys)

    z0p = jnp.transpose(z0, (2, 0, 1))[order]
    pip = pi_pos.T[order]
    byp = beta[y, :].T[order]
    lpp = linear_pi.T[order]
    btp = jnp.pad(beta.T[order], ((0, 0), (0, CP - C)))
    e2 = jnp.exp(2.0 * btp)
    em2 = jnp.exp(-2.0 * btp)
    btT = jnp.pad(beta.T, ((0, 0), (0, CP - C)))
    b0 = jnp.full((1, CP), -1e30, jnp.float32).at[0, :C].set(beta_0)
    yo = jax.nn.one_hot(y, CP, dtype=jnp.float32)

    devs = jax.devices()
    nd = 2 if len(devs) >= 2 else 1
    mesh = jax.sharding.Mesh(np.array(devs[:nd]), ("d",))
    P = jax.sharding.PartitionSpec

    def _run(z0_, z0p_, u_, pip_, byp_, lpp_, e2_, em2_, btT_, b0_, yo_):
        nb_loc = z0_.shape[1] // NB
        return pl.pallas_call(
            _gibbs_body,
            out_shape=jax.ShapeDtypeStruct((nb_loc, 1, 128), jnp.float32),
            grid=(nb_loc,),
            in_specs=[
                pl.BlockSpec((S, NB, M), lambda i: (0, i, 0)),
                pl.BlockSpec((M, S, NB), lambda i: (0, 0, i)),
                pl.BlockSpec((M, S, NB), lambda i: (0, 0, i)),
                pl.BlockSpec((M, NB), lambda i: (0, i)),
                pl.BlockSpec((M, NB), lambda i: (0, i)),
                pl.BlockSpec((M, NB), lambda i: (0, i)),
                pl.BlockSpec((M, CP), lambda i: (0, 0)),
                pl.BlockSpec((M, CP), lambda i: (0, 0)),
                pl.BlockSpec((M, CP), lambda i: (0, 0)),
                pl.BlockSpec((1, CP), lambda i: (0, 0)),
                pl.BlockSpec((NB, CP), lambda i: (i, 0)),
            ],
            out_specs=pl.BlockSpec((1, 1, 128), lambda i: (i, 0, 0)),
            scratch_shapes=[pltpu.VMEM((S, CP, NB), jnp.float32)],
            compiler_params=pltpu.CompilerParams(
                dimension_semantics=("parallel",),
                vmem_limit_bytes=48 * 1024 * 1024,
            ),
            name="lb_cnn_gibbs",
            interpret=False,
        )(z0_, z0p_, u_, pip_, byp_, lpp_, e2_, em2_, btT_, b0_, yo_)

    sharded = jax.shard_map(
        _run,
        mesh=mesh,
        in_specs=(
            P(None, "d", None),
            P(None, None, "d"),
            P(None, None, "d"),
            P(None, "d"),
            P(None, "d"),
            P(None, "d"),
            P(None, None),
            P(None, None),
            P(None, None),
            P(None, None),
            P("d", None),
        ),
        out_specs=P("d", None, None),
        check_vma=False,
    )
    out = sharded(z0, z0p, u, pip, byp, lpp, e2, em2, btT, b0, yo)
    return jnp.sum(out[:, 0, 0])
